# baseline (speedup 1.0000x reference)
import functools

import jax
import jax.numpy as jnp
from jax import lax
from jax.experimental import pallas as pl
from jax.experimental.pallas import tpu as pltpu
from jax.experimental.pallas import tpu_sc as plsc

NUM_CORES = 2
NUM_SUBCORES = 16
NUM_WORKERS = NUM_CORES * NUM_SUBCORES
CHUNK = 128


@functools.lru_cache(maxsize=None)
def _make_gather(num_rows, dim, batch):
    assert batch % (NUM_WORKERS * CHUNK) == 0
    n_chunks = batch // (NUM_WORKERS * CHUNK)
    rows_per_worker = n_chunks * CHUNK
    mesh = plsc.VectorSubcoreMesh(core_axis_name="c", subcore_axis_name="s")

    @functools.partial(
        pl.kernel,
        mesh=mesh,
        out_type=jax.ShapeDtypeStruct((batch, dim), jnp.float32),
        compiler_params=pltpu.CompilerParams(use_tc_tiling_on_sc=False),
        scratch_types=[
            pltpu.VMEM((n_chunks, CHUNK), jnp.int32),
            pltpu.VMEM((CHUNK, dim), jnp.float32),
            pltpu.SemaphoreType.DMA,
        ],
    )
    def gather_kernel(idx_hbm, table_hbm, out_hbm, idx_v, buf, sem):
        wid = lax.axis_index("s") * NUM_CORES + lax.axis_index("c")
        base = wid * rows_per_worker
        pltpu.sync_copy(idx_hbm.at[wid], idx_v)

        def body(g, carry):
            pltpu.async_copy(table_hbm.at[idx_v.at[g]], buf, sem).wait()
            pltpu.sync_copy(buf, out_hbm.at[pl.ds(base + g * CHUNK, CHUNK)])
            return carry

        lax.fori_loop(0, n_chunks, body, 0)

    return gather_kernel


def kernel(input, weight):
    b0, b1 = input.shape
    num_rows, dim = weight.shape
    idx = input.reshape(NUM_WORKERS, -1, CHUNK).astype(jnp.int32)
    out = _make_gather(num_rows, dim, b0 * b1)(idx, weight)
    return out.reshape(b0, b1, dim)


# speedup vs baseline: 1.0815x; 1.0815x over previous
import functools

import jax
import jax.numpy as jnp
from jax import lax
from jax.experimental import pallas as pl
from jax.experimental.pallas import tpu as pltpu
from jax.experimental.pallas import tpu_sc as plsc

NUM_CORES = 2
NUM_SUBCORES = 16
NUM_WORKERS = NUM_CORES * NUM_SUBCORES
CHUNK = 128
NBUF = 8
LOOKAHEAD = 4


@functools.lru_cache(maxsize=None)
def _make_gather(num_rows, dim, batch):
    assert batch % (NUM_WORKERS * CHUNK) == 0
    n_chunks = batch // (NUM_WORKERS * CHUNK)
    assert n_chunks % NBUF == 0 and n_chunks >= 2 * NBUF
    rows_per_worker = n_chunks * CHUNK
    mesh = plsc.VectorSubcoreMesh(core_axis_name="c", subcore_axis_name="s")

    @functools.partial(
        pl.kernel,
        mesh=mesh,
        out_type=jax.ShapeDtypeStruct((batch, dim), jnp.float32),
        compiler_params=pltpu.CompilerParams(use_tc_tiling_on_sc=False),
        scratch_types=[
            pltpu.VMEM((n_chunks, CHUNK), jnp.int32),
            pltpu.VMEM((NBUF, CHUNK, dim), jnp.float32),
            pltpu.SemaphoreType.DMA((NBUF,)),
            pltpu.SemaphoreType.DMA((NBUF,)),
        ],
    )
    def gather_kernel(idx_hbm, table_hbm, out_hbm, idx_v, bufs, gsem, osem):
        wid = lax.axis_index("s") * NUM_CORES + lax.axis_index("c")
        base = wid * rows_per_worker
        pltpu.sync_copy(idx_hbm.at[wid], idx_v)

        def gather_start(g, slot):
            pltpu.async_copy(table_hbm.at[idx_v.at[g]], bufs.at[slot],
                             gsem.at[slot])

        def gather_wait(g, slot):
            pltpu.make_async_copy(table_hbm.at[idx_v.at[g]], bufs.at[slot],
                                  gsem.at[slot]).wait()

        def out_start(g, slot):
            pltpu.async_copy(bufs.at[slot],
                             out_hbm.at[pl.ds(base + g * CHUNK, CHUNK)],
                             osem.at[slot])

        def out_wait(g, slot):
            pltpu.make_async_copy(bufs.at[slot],
                                  out_hbm.at[pl.ds(base + g * CHUNK, CHUNK)],
                                  osem.at[slot]).wait()

        for s in range(LOOKAHEAD):
            gather_start(s, s)
        for g in range(NBUF - LOOKAHEAD):
            gather_start(g + LOOKAHEAD, g + LOOKAHEAD)
            gather_wait(g, g)
            out_start(g, g)

        start = NBUF - LOOKAHEAD
        n_main = n_chunks - NBUF

        def body(t, carry):
            for s in range(NBUF):
                g = start + t * NBUF + s
                slot = (start + s) % NBUF
                f = g + LOOKAHEAD
                fslot = (slot + LOOKAHEAD) % NBUF
                out_wait(f - NBUF, fslot)
                gather_start(f, fslot)
                gather_wait(g, slot)
                out_start(g, slot)
            return carry

        lax.fori_loop(0, n_main // NBUF, body, 0)

        for k in range(LOOKAHEAD):
            g = n_chunks - LOOKAHEAD + k
            slot = g % NBUF
            gather_wait(g, slot)
            out_start(g, slot)
        for k in range(NBUF):
            g = n_chunks - NBUF + k
            out_wait(g, g % NBUF)

    return gather_kernel


def kernel(input, weight):
    b0, b1 = input.shape
    num_rows, dim = weight.shape
    idx = input.reshape(NUM_WORKERS, -1, CHUNK).astype(jnp.int32)
    out = _make_gather(num_rows, dim, b0 * b1)(idx, weight)
    return out.reshape(b0, b1, dim)
